# baseline (speedup 1.0000x reference)
import dataclasses
import functools

import jax
import jax.numpy as jnp
from jax import lax
from jax.experimental import pallas as pl
from jax.experimental.pallas import tpu as pltpu
from jax.experimental.pallas import tpu_sc as plsc

_NC = 2
_NS = 16
_L = 16
_NW = _NC * _NS
_CHUNK = 96


_NBUF = 3


def _lane_broadcast(vec, lane):
    return lax.gather(
        vec, jnp.full((_L, 1), lane, jnp.int32),
        lax.GatherDimensionNumbers(
            offset_dims=(), collapsed_slice_dims=(0,), start_index_map=(0,)),
        slice_sizes=(1,),
        mode=lax.GatherScatterMode.PROMISE_IN_BOUNDS)


@functools.cache
def _make_spmm(n_nodes_p: int, d: int, n_edges_p: int):
    assert n_nodes_p % (_NS * 64) == 0
    assert n_edges_p % (_NW * _CHUNK * _NBUF) == 0
    epw = n_edges_p // _NW
    n_chunks = epw // _CHUNK
    assert n_chunks % _NBUF == 0 and n_chunks >= 4
    rows_pt = n_nodes_p // _NS
    seg = 3 * _CHUNK
    mesh = plsc.VectorSubcoreMesh(core_axis_name="c", subcore_axis_name="s")

    def body(h_hbm, meta_hbm, out_hbm,
             msg0, msg1, msg2, ix0, ix1, ix2, rs0, rs1, rs2, acc_sh,
             g0, g1, g2, s0, s1, s2, i0, i1, i2):
        msgs = (msg0, msg1, msg2)
        idxs = (ix0, ix1, ix2)
        rstg = (rs0, rs1, rs2)
        gsem = (g0, g1, g2)
        ssem = (s0, s1, s2)
        isem = (i0, i1, i2)
        c = lax.axis_index("c")
        s = lax.axis_index("s")
        wid = c * _NS + s
        cbase = wid * n_chunks

        zeros = jnp.zeros((_L,), jnp.float32)

        @pl.loop(0, 64)
        def _(i):
            for j in range(d // _L):
                msg0[i, pl.ds(j * _L, _L)] = zeros

        @pl.loop(0, rows_pt // 64)
        def _(i):
            pltpu.sync_copy(
                msg0.at[pl.ds(0, 64)],
                acc_sh.at[pl.ds(s * rows_pt + i * 64, 64)])

        plsc.subcore_barrier()

        for b in range(_NBUF):
            pltpu.async_copy(
                meta_hbm.at[pl.ds((cbase + b) * seg, seg)], idxs[b], isem[b])
        for b in range(2):
            pltpu.make_async_copy(
                meta_hbm.at[pl.ds((cbase + b) * seg, seg)], idxs[b],
                isem[b]).wait()
            pltpu.async_copy(
                h_hbm.at[idxs[b].at[pl.ds(0, _CHUNK)]], msgs[b], gsem[b])

        @pl.loop(0, n_chunks, step=_NBUF)
        def _(k):
            for b in range(_NBUF):
                kk = k + b
                buf = msgs[b]
                meta = idxs[b]
                bp = (b + _NBUF - 1) % _NBUF
                pltpu.make_async_copy(
                    h_hbm.at[meta.at[pl.ds(0, _CHUNK)]], buf,
                    gsem[b]).wait()

                for j in range(_CHUNK // _L):
                    rstg[b][pl.ds(j * _L, _L)] = (
                        meta[pl.ds(_CHUNK + j * _L, _L)])

                @pl.when(kk >= 1)
                def _():
                    pltpu.make_async_copy(
                        msgs[bp], acc_sh.at[rstg[bp]], ssem[bp]).wait()

                @pl.when(kk + 2 < n_chunks)
                def _():
                    pltpu.make_async_copy(
                        meta_hbm.at[pl.ds((cbase + kk + 2) * seg, seg)],
                        idxs[bp], isem[bp]).wait()
                    pltpu.async_copy(
                        h_hbm.at[idxs[bp].at[pl.ds(0, _CHUNK)]], msgs[bp],
                        gsem[bp])

                @pl.loop(0, _CHUNK)
                def _(e):
                    iv = jnp.broadcast_to(2 * _CHUNK + e, (_L,))
                    wb = plsc.bitcast(
                        plsc.load_gather(meta, [iv]), jnp.float32)
                    for j in range(d // _L):
                        sl = (e, pl.ds(j * _L, _L))
                        buf[sl] = buf[sl] * wb

                pltpu.async_copy(buf, acc_sh.at[rstg[b]], ssem[b],
                                 add=True)

                @pl.when(kk + 3 < n_chunks)
                def _():
                    pltpu.async_copy(
                        meta_hbm.at[pl.ds((cbase + kk + 3) * seg, seg)],
                        idxs[b], isem[b])

        bl = (n_chunks - 1) % _NBUF
        pltpu.make_async_copy(
            msgs[bl], acc_sh.at[rstg[bl]], ssem[bl]).wait()

        plsc.subcore_barrier()

        @pl.loop(0, rows_pt // 64)
        def _(i):
            r0 = s * rows_pt + i * 64
            pltpu.sync_copy(acc_sh.at[pl.ds(r0, 64)],
                            out_hbm.at[c, pl.ds(r0, 64)])

    cp = pltpu.CompilerParams()
    if "needs_layout_passes" in pltpu.CompilerParams.__dataclass_fields__:
        cp = dataclasses.replace(cp, needs_layout_passes=False)
    return pl.kernel(
        body,
        out_type=jax.ShapeDtypeStruct((_NC, n_nodes_p, d), jnp.float32),
        mesh=mesh,
        compiler_params=cp,
        scratch_types=[
            pltpu.VMEM((_CHUNK, 128), jnp.float32),
            pltpu.VMEM((_CHUNK, 128), jnp.float32),
            pltpu.VMEM((_CHUNK, 128), jnp.float32),
            pltpu.VMEM((3 * _CHUNK,), jnp.int32),
            pltpu.VMEM((3 * _CHUNK,), jnp.int32),
            pltpu.VMEM((3 * _CHUNK,), jnp.int32),
            pltpu.VMEM((_CHUNK,), jnp.int32),
            pltpu.VMEM((_CHUNK,), jnp.int32),
            pltpu.VMEM((_CHUNK,), jnp.int32),
            pltpu.VMEM_SHARED((n_nodes_p, 128), jnp.float32),
            pltpu.SemaphoreType.DMA,
            pltpu.SemaphoreType.DMA,
            pltpu.SemaphoreType.DMA,
            pltpu.SemaphoreType.DMA,
            pltpu.SemaphoreType.DMA,
            pltpu.SemaphoreType.DMA,
            pltpu.SemaphoreType.DMA,
            pltpu.SemaphoreType.DMA,
            pltpu.SemaphoreType.DMA,
        ],
    )


@functools.cache
def _make_tc_layer1(n_p: int, d: int, blk: int):
    def body(p_ref, w_ref, o_ref):
        a = p_ref[0] + p_ref[1]
        o_ref[...] = jnp.dot(a, w_ref[...], preferred_element_type=jnp.float32)

    return pl.pallas_call(
        body,
        grid=(n_p // blk,),
        in_specs=[pl.BlockSpec((2, blk, d), lambda i: (0, i, 0)),
                  pl.BlockSpec((d, d), lambda i: (0, 0))],
        out_specs=pl.BlockSpec((blk, d), lambda i: (i, 0)),
        out_shape=jax.ShapeDtypeStruct((n_p, d), jnp.float32),
    )


@functools.cache
def _make_tc_layer2(n_p: int, d: int, blk: int):
    def body(x_ref, h1_ref, q_ref, w_ref, o_ref):
        a = q_ref[0] + q_ref[1]
        h2 = jnp.dot(a, w_ref[...], preferred_element_type=jnp.float32)
        o_ref[...] = (x_ref[...] + h1_ref[...] + h2) * (1.0 / 3.0)

    return pl.pallas_call(
        body,
        grid=(n_p // blk,),
        in_specs=[pl.BlockSpec((blk, d), lambda i: (i, 0)),
                  pl.BlockSpec((blk, d), lambda i: (i, 0)),
                  pl.BlockSpec((2, blk, d), lambda i: (0, i, 0)),
                  pl.BlockSpec((d, d), lambda i: (0, 0))],
        out_specs=pl.BlockSpec((blk, d), lambda i: (i, 0)),
        out_shape=jax.ShapeDtypeStruct((n_p, d), jnp.float32),
    )


def kernel(x, edge_index, edge_weight, W0, W1):
    n, d = x.shape
    e = edge_weight.shape[0]
    n_p = ((n + _NS * 64 - 1) // (_NS * 64)) * (_NS * 64)
    e_quant = _NW * _CHUNK * _NBUF
    e_p = ((e + e_quant - 1) // e_quant) * e_quant

    row = edge_index[0].astype(jnp.int32)
    col = edge_index[1].astype(jnp.int32)
    w = edge_weight.astype(jnp.float32)
    pad_e = e_p - e
    if pad_e:
        row = jnp.concatenate([row, jnp.zeros((pad_e,), jnp.int32)])
        col = jnp.concatenate([col, jnp.zeros((pad_e,), jnp.int32)])
        w = jnp.concatenate([w, jnp.zeros((pad_e,), jnp.float32)])
    wbits = lax.bitcast_convert_type(w, jnp.int32)
    meta = jnp.concatenate(
        [col.reshape(-1, _CHUNK), row.reshape(-1, _CHUNK),
         wbits.reshape(-1, _CHUNK)], axis=1).reshape(-1)
    x_p = x
    if n_p != n:
        x_p = jnp.concatenate([x, jnp.zeros((n_p - n, d), x.dtype)])

    spmm = _make_spmm(n_p, d, e_p)
    tc1 = _make_tc_layer1(n_p, d, 1024)
    tc2 = _make_tc_layer2(n_p, d, 1024)

    agg1 = spmm(x_p, meta)
    h1 = tc1(agg1, W0)
    agg2 = spmm(h1, meta)
    out_p = tc2(x_p, h1, agg2, W1)
    return out_p[:n]


# speedup vs baseline: 1.1039x; 1.1039x over previous
import dataclasses
import functools

import jax
import jax.numpy as jnp
from jax import lax
from jax.experimental import pallas as pl
from jax.experimental.pallas import tpu as pltpu
from jax.experimental.pallas import tpu_sc as plsc

_NC = 2
_NS = 16
_L = 16
_NW = _NC * _NS
_CHUNK = 96


_NBUF = 3


def _lane_broadcast(vec, lane):
    return lax.gather(
        vec, jnp.full((_L, 1), lane, jnp.int32),
        lax.GatherDimensionNumbers(
            offset_dims=(), collapsed_slice_dims=(0,), start_index_map=(0,)),
        slice_sizes=(1,),
        mode=lax.GatherScatterMode.PROMISE_IN_BOUNDS)


@functools.cache
def _make_spmm(n_nodes_p: int, d: int, n_edges_p: int):
    assert n_nodes_p % (_NS * 64) == 0
    assert n_edges_p % (_NW * _CHUNK * _NBUF) == 0
    epw = n_edges_p // _NW
    n_chunks = epw // _CHUNK
    assert n_chunks % _NBUF == 0 and n_chunks >= 4
    rows_pt = n_nodes_p // _NS
    seg = 3 * _CHUNK
    mesh = plsc.VectorSubcoreMesh(core_axis_name="c", subcore_axis_name="s")

    def body(h_hbm, meta_hbm, out_hbm,
             msg0, msg1, msg2, ix0, ix1, ix2, rs0, rs1, rs2, acc_sh,
             g0, g1, g2, s0, s1, s2, i0, i1, i2):
        msgs = (msg0, msg1, msg2)
        idxs = (ix0, ix1, ix2)
        rstg = (rs0, rs1, rs2)
        gsem = (g0, g1, g2)
        ssem = (s0, s1, s2)
        isem = (i0, i1, i2)
        c = lax.axis_index("c")
        s = lax.axis_index("s")
        wid = c * _NS + s
        cbase = wid * n_chunks

        zeros = jnp.zeros((_L,), jnp.float32)

        @pl.loop(0, 64)
        def _(i):
            for j in range(d // _L):
                msg0[i, pl.ds(j * _L, _L)] = zeros

        @pl.loop(0, rows_pt // 64)
        def _(i):
            pltpu.sync_copy(
                msg0.at[pl.ds(0, 64)],
                acc_sh.at[pl.ds(s * rows_pt + i * 64, 64)])

        plsc.subcore_barrier()

        for b in range(_NBUF):
            pltpu.async_copy(
                meta_hbm.at[pl.ds((cbase + b) * seg, seg)], idxs[b], isem[b])
        for b in range(2):
            pltpu.make_async_copy(
                meta_hbm.at[pl.ds((cbase + b) * seg, seg)], idxs[b],
                isem[b]).wait()
            pltpu.async_copy(
                h_hbm.at[idxs[b].at[pl.ds(0, _CHUNK)]], msgs[b], gsem[b])

        @pl.loop(0, n_chunks, step=_NBUF)
        def _(k):
            for b in range(_NBUF):
                kk = k + b
                buf = msgs[b]
                meta = idxs[b]
                bp = (b + _NBUF - 1) % _NBUF
                pltpu.make_async_copy(
                    h_hbm.at[meta.at[pl.ds(0, _CHUNK)]], buf,
                    gsem[b]).wait()

                for j in range(_CHUNK // _L):
                    rstg[b][pl.ds(j * _L, _L)] = (
                        meta[pl.ds(_CHUNK + j * _L, _L)])

                @pl.loop(0, _CHUNK, unroll=4)
                def _(e):
                    iv = jnp.broadcast_to(2 * _CHUNK + e, (_L,))
                    wb = plsc.bitcast(
                        plsc.load_gather(meta, [iv]), jnp.float32)
                    for j in range(d // _L):
                        sl = (e, pl.ds(j * _L, _L))
                        buf[sl] = buf[sl] * wb

                pltpu.async_copy(buf, acc_sh.at[rstg[b]], ssem[b],
                                 add=True)

                @pl.when(kk >= 1)
                def _():
                    pltpu.make_async_copy(
                        msgs[bp], acc_sh.at[rstg[bp]], ssem[bp]).wait()

                @pl.when(kk + 2 < n_chunks)
                def _():
                    pltpu.make_async_copy(
                        meta_hbm.at[pl.ds((cbase + kk + 2) * seg, seg)],
                        idxs[bp], isem[bp]).wait()
                    pltpu.async_copy(
                        h_hbm.at[idxs[bp].at[pl.ds(0, _CHUNK)]], msgs[bp],
                        gsem[bp])

                @pl.when(kk + 3 < n_chunks)
                def _():
                    pltpu.async_copy(
                        meta_hbm.at[pl.ds((cbase + kk + 3) * seg, seg)],
                        idxs[b], isem[b])

        bl = (n_chunks - 1) % _NBUF
        pltpu.make_async_copy(
            msgs[bl], acc_sh.at[rstg[bl]], ssem[bl]).wait()

        plsc.subcore_barrier()

        @pl.loop(0, rows_pt // 64)
        def _(i):
            r0 = s * rows_pt + i * 64
            pltpu.sync_copy(acc_sh.at[pl.ds(r0, 64)],
                            out_hbm.at[c, pl.ds(r0, 64)])

    cp = pltpu.CompilerParams()
    if "needs_layout_passes" in pltpu.CompilerParams.__dataclass_fields__:
        cp = dataclasses.replace(cp, needs_layout_passes=False)
    return pl.kernel(
        body,
        out_type=jax.ShapeDtypeStruct((_NC, n_nodes_p, d), jnp.float32),
        mesh=mesh,
        compiler_params=cp,
        scratch_types=[
            pltpu.VMEM((_CHUNK, 128), jnp.float32),
            pltpu.VMEM((_CHUNK, 128), jnp.float32),
            pltpu.VMEM((_CHUNK, 128), jnp.float32),
            pltpu.VMEM((3 * _CHUNK,), jnp.int32),
            pltpu.VMEM((3 * _CHUNK,), jnp.int32),
            pltpu.VMEM((3 * _CHUNK,), jnp.int32),
            pltpu.VMEM((_CHUNK,), jnp.int32),
            pltpu.VMEM((_CHUNK,), jnp.int32),
            pltpu.VMEM((_CHUNK,), jnp.int32),
            pltpu.VMEM_SHARED((n_nodes_p, 128), jnp.float32),
            pltpu.SemaphoreType.DMA,
            pltpu.SemaphoreType.DMA,
            pltpu.SemaphoreType.DMA,
            pltpu.SemaphoreType.DMA,
            pltpu.SemaphoreType.DMA,
            pltpu.SemaphoreType.DMA,
            pltpu.SemaphoreType.DMA,
            pltpu.SemaphoreType.DMA,
            pltpu.SemaphoreType.DMA,
        ],
    )


@functools.cache
def _make_tc_layer1(n_p: int, d: int, blk: int):
    def body(p_ref, w_ref, o_ref):
        a = p_ref[0] + p_ref[1]
        o_ref[...] = jnp.dot(a, w_ref[...], preferred_element_type=jnp.float32)

    return pl.pallas_call(
        body,
        grid=(n_p // blk,),
        in_specs=[pl.BlockSpec((2, blk, d), lambda i: (0, i, 0)),
                  pl.BlockSpec((d, d), lambda i: (0, 0))],
        out_specs=pl.BlockSpec((blk, d), lambda i: (i, 0)),
        out_shape=jax.ShapeDtypeStruct((n_p, d), jnp.float32),
    )


@functools.cache
def _make_tc_layer2(n_p: int, d: int, blk: int):
    def body(x_ref, h1_ref, q_ref, w_ref, o_ref):
        a = q_ref[0] + q_ref[1]
        h2 = jnp.dot(a, w_ref[...], preferred_element_type=jnp.float32)
        o_ref[...] = (x_ref[...] + h1_ref[...] + h2) * (1.0 / 3.0)

    return pl.pallas_call(
        body,
        grid=(n_p // blk,),
        in_specs=[pl.BlockSpec((blk, d), lambda i: (i, 0)),
                  pl.BlockSpec((blk, d), lambda i: (i, 0)),
                  pl.BlockSpec((2, blk, d), lambda i: (0, i, 0)),
                  pl.BlockSpec((d, d), lambda i: (0, 0))],
        out_specs=pl.BlockSpec((blk, d), lambda i: (i, 0)),
        out_shape=jax.ShapeDtypeStruct((n_p, d), jnp.float32),
    )


def kernel(x, edge_index, edge_weight, W0, W1):
    n, d = x.shape
    e = edge_weight.shape[0]
    n_p = ((n + _NS * 64 - 1) // (_NS * 64)) * (_NS * 64)
    e_quant = _NW * _CHUNK * _NBUF
    e_p = ((e + e_quant - 1) // e_quant) * e_quant

    row = edge_index[0].astype(jnp.int32)
    col = edge_index[1].astype(jnp.int32)
    w = edge_weight.astype(jnp.float32)
    pad_e = e_p - e
    if pad_e:
        row = jnp.concatenate([row, jnp.zeros((pad_e,), jnp.int32)])
        col = jnp.concatenate([col, jnp.zeros((pad_e,), jnp.int32)])
        w = jnp.concatenate([w, jnp.zeros((pad_e,), jnp.float32)])
    wbits = lax.bitcast_convert_type(w, jnp.int32)
    meta = jnp.concatenate(
        [col.reshape(-1, _CHUNK), row.reshape(-1, _CHUNK),
         wbits.reshape(-1, _CHUNK)], axis=1).reshape(-1)
    x_p = x
    if n_p != n:
        x_p = jnp.concatenate([x, jnp.zeros((n_p - n, d), x.dtype)])

    spmm = _make_spmm(n_p, d, e_p)
    tc1 = _make_tc_layer1(n_p, d, 1024)
    tc2 = _make_tc_layer2(n_p, d, 1024)

    agg1 = spmm(x_p, meta)
    h1 = tc1(agg1, W0)
    agg2 = spmm(h1, meta)
    out_p = tc2(x_p, h1, agg2, W1)
    return out_p[:n]


# speedup vs baseline: 1.1968x; 1.0841x over previous
import dataclasses
import functools

import jax
import jax.numpy as jnp
from jax import lax
from jax.experimental import pallas as pl
from jax.experimental.pallas import tpu as pltpu
from jax.experimental.pallas import tpu_sc as plsc

_NC = 2
_NS = 16
_L = 16
_NW = _NC * _NS
_CHUNK = 96


_NBUF = 3
_C0_FRAC = 0.63


def _lane_broadcast(vec, lane):
    return lax.gather(
        vec, jnp.full((_L, 1), lane, jnp.int32),
        lax.GatherDimensionNumbers(
            offset_dims=(), collapsed_slice_dims=(0,), start_index_map=(0,)),
        slice_sizes=(1,),
        mode=lax.GatherScatterMode.PROMISE_IN_BOUNDS)


@functools.cache
def _make_spmm(n_nodes_p: int, d: int, n_edges_p: int,
               c0_frac: float = 0.5):
    assert n_nodes_p % (_NS * 64) == 0
    assert n_edges_p % (_NW * _CHUNK * _NBUF) == 0
    tot_chunks = n_edges_p // (_NS * _CHUNK)
    ca = max(_NBUF * 2, int(round(tot_chunks * c0_frac / _NBUF)) * _NBUF)
    cb = tot_chunks - ca
    assert cb >= _NBUF * 2 and ca % _NBUF == 0 and cb % _NBUF == 0
    rows_pt = n_nodes_p // _NS
    seg = 3 * _CHUNK
    mesh = plsc.VectorSubcoreMesh(core_axis_name="c", subcore_axis_name="s")

    def body(h_hbm, meta_hbm, out_hbm,
             msg0, msg1, msg2, ix0, ix1, ix2, rs0, rs1, rs2, acc_sh,
             g0, g1, g2, s0, s1, s2, i0, i1, i2):
        msgs = (msg0, msg1, msg2)
        idxs = (ix0, ix1, ix2)
        rstg = (rs0, rs1, rs2)
        gsem = (g0, g1, g2)
        ssem = (s0, s1, s2)
        isem = (i0, i1, i2)
        c = lax.axis_index("c")
        s = lax.axis_index("s")
        cbase = jnp.where(c == 0, s * ca, _NS * ca + s * cb)

        zeros = jnp.zeros((_L,), jnp.float32)

        @pl.loop(0, 64)
        def _(i):
            for j in range(d // _L):
                msg0[i, pl.ds(j * _L, _L)] = zeros

        @pl.loop(0, rows_pt // 64)
        def _(i):
            pltpu.sync_copy(
                msg0.at[pl.ds(0, 64)],
                acc_sh.at[pl.ds(s * rows_pt + i * 64, 64)])

        plsc.subcore_barrier()

        for b in range(_NBUF):
            pltpu.async_copy(
                meta_hbm.at[pl.ds((cbase + b) * seg, seg)], idxs[b], isem[b])
        for b in range(2):
            pltpu.make_async_copy(
                meta_hbm.at[pl.ds((cbase + b) * seg, seg)], idxs[b],
                isem[b]).wait()
            pltpu.async_copy(
                h_hbm.at[idxs[b].at[pl.ds(0, _CHUNK)]], msgs[b], gsem[b])

        def main_loop(n_chunks):
            @pl.loop(0, n_chunks, step=_NBUF)
            def _(k):
                for b in range(_NBUF):
                    kk = k + b
                    buf = msgs[b]
                    meta = idxs[b]
                    bp = (b + _NBUF - 1) % _NBUF
                    pltpu.make_async_copy(
                        h_hbm.at[meta.at[pl.ds(0, _CHUNK)]], buf,
                        gsem[b]).wait()

                    for j in range(_CHUNK // _L):
                        rstg[b][pl.ds(j * _L, _L)] = (
                            meta[pl.ds(_CHUNK + j * _L, _L)])

                    @pl.loop(0, _CHUNK, unroll=4)
                    def _(e):
                        iv = jnp.broadcast_to(2 * _CHUNK + e, (_L,))
                        wb = plsc.bitcast(
                            plsc.load_gather(meta, [iv]), jnp.float32)
                        for j in range(d // _L):
                            sl = (e, pl.ds(j * _L, _L))
                            buf[sl] = buf[sl] * wb

                    pltpu.async_copy(buf, acc_sh.at[rstg[b]], ssem[b],
                                     add=True)

                    @pl.when(kk >= 1)
                    def _():
                        pltpu.make_async_copy(
                            msgs[bp], acc_sh.at[rstg[bp]], ssem[bp]).wait()

                    @pl.when(kk + 2 < n_chunks)
                    def _():
                        pltpu.make_async_copy(
                            meta_hbm.at[pl.ds((cbase + kk + 2) * seg, seg)],
                            idxs[bp], isem[bp]).wait()
                        pltpu.async_copy(
                            h_hbm.at[idxs[bp].at[pl.ds(0, _CHUNK)]],
                            msgs[bp], gsem[bp])

                    @pl.when(kk + 3 < n_chunks)
                    def _():
                        pltpu.async_copy(
                            meta_hbm.at[pl.ds((cbase + kk + 3) * seg, seg)],
                            idxs[b], isem[b])

            bl = (n_chunks - 1) % _NBUF
            pltpu.make_async_copy(
                msgs[bl], acc_sh.at[rstg[bl]], ssem[bl]).wait()

        if ca == cb:
            main_loop(ca)
        else:
            @pl.when(c == 0)
            def _():
                main_loop(ca)

            @pl.when(c == 1)
            def _():
                main_loop(cb)

        plsc.subcore_barrier()

        @pl.loop(0, rows_pt // 64)
        def _(i):
            r0 = s * rows_pt + i * 64
            pltpu.sync_copy(acc_sh.at[pl.ds(r0, 64)],
                            out_hbm.at[c, pl.ds(r0, 64)])

    cp = pltpu.CompilerParams()
    if "needs_layout_passes" in pltpu.CompilerParams.__dataclass_fields__:
        cp = dataclasses.replace(cp, needs_layout_passes=False)
    return pl.kernel(
        body,
        out_type=jax.ShapeDtypeStruct((_NC, n_nodes_p, d), jnp.float32),
        mesh=mesh,
        compiler_params=cp,
        scratch_types=[
            pltpu.VMEM((_CHUNK, 128), jnp.float32),
            pltpu.VMEM((_CHUNK, 128), jnp.float32),
            pltpu.VMEM((_CHUNK, 128), jnp.float32),
            pltpu.VMEM((3 * _CHUNK,), jnp.int32),
            pltpu.VMEM((3 * _CHUNK,), jnp.int32),
            pltpu.VMEM((3 * _CHUNK,), jnp.int32),
            pltpu.VMEM((_CHUNK,), jnp.int32),
            pltpu.VMEM((_CHUNK,), jnp.int32),
            pltpu.VMEM((_CHUNK,), jnp.int32),
            pltpu.VMEM_SHARED((n_nodes_p, 128), jnp.float32),
            pltpu.SemaphoreType.DMA,
            pltpu.SemaphoreType.DMA,
            pltpu.SemaphoreType.DMA,
            pltpu.SemaphoreType.DMA,
            pltpu.SemaphoreType.DMA,
            pltpu.SemaphoreType.DMA,
            pltpu.SemaphoreType.DMA,
            pltpu.SemaphoreType.DMA,
            pltpu.SemaphoreType.DMA,
        ],
    )


@functools.cache
def _make_tc_layer1(n_p: int, d: int, blk: int):
    def body(p_ref, w_ref, o_ref):
        a = p_ref[0] + p_ref[1]
        o_ref[...] = jnp.dot(a, w_ref[...], preferred_element_type=jnp.float32)

    return pl.pallas_call(
        body,
        grid=(n_p // blk,),
        in_specs=[pl.BlockSpec((2, blk, d), lambda i: (0, i, 0)),
                  pl.BlockSpec((d, d), lambda i: (0, 0))],
        out_specs=pl.BlockSpec((blk, d), lambda i: (i, 0)),
        out_shape=jax.ShapeDtypeStruct((n_p, d), jnp.float32),
    )


@functools.cache
def _make_tc_layer2(n_p: int, d: int, blk: int):
    def body(x_ref, h1_ref, q_ref, w_ref, o_ref):
        a = q_ref[0] + q_ref[1]
        h2 = jnp.dot(a, w_ref[...], preferred_element_type=jnp.float32)
        o_ref[...] = (x_ref[...] + h1_ref[...] + h2) * (1.0 / 3.0)

    return pl.pallas_call(
        body,
        grid=(n_p // blk,),
        in_specs=[pl.BlockSpec((blk, d), lambda i: (i, 0)),
                  pl.BlockSpec((blk, d), lambda i: (i, 0)),
                  pl.BlockSpec((2, blk, d), lambda i: (0, i, 0)),
                  pl.BlockSpec((d, d), lambda i: (0, 0))],
        out_specs=pl.BlockSpec((blk, d), lambda i: (i, 0)),
        out_shape=jax.ShapeDtypeStruct((n_p, d), jnp.float32),
    )


def kernel(x, edge_index, edge_weight, W0, W1):
    n, d = x.shape
    e = edge_weight.shape[0]
    n_p = ((n + _NS * 64 - 1) // (_NS * 64)) * (_NS * 64)
    e_quant = _NW * _CHUNK * _NBUF
    e_p = ((e + e_quant - 1) // e_quant) * e_quant

    row = edge_index[0].astype(jnp.int32)
    col = edge_index[1].astype(jnp.int32)
    w = edge_weight.astype(jnp.float32)
    pad_e = e_p - e
    if pad_e:
        row = jnp.concatenate([row, jnp.zeros((pad_e,), jnp.int32)])
        col = jnp.concatenate([col, jnp.zeros((pad_e,), jnp.int32)])
        w = jnp.concatenate([w, jnp.zeros((pad_e,), jnp.float32)])
    wbits = lax.bitcast_convert_type(w, jnp.int32)
    meta = jnp.concatenate(
        [col.reshape(-1, _CHUNK), row.reshape(-1, _CHUNK),
         wbits.reshape(-1, _CHUNK)], axis=1).reshape(-1)
    x_p = x
    if n_p != n:
        x_p = jnp.concatenate([x, jnp.zeros((n_p - n, d), x.dtype)])

    spmm = _make_spmm(n_p, d, e_p, _C0_FRAC)
    tc1 = _make_tc_layer1(n_p, d, 1024)
    tc2 = _make_tc_layer2(n_p, d, 1024)

    agg1 = spmm(x_p, meta)
    h1 = tc1(agg1, W0)
    agg2 = spmm(h1, meta)
    out_p = tc2(x_p, h1, agg2, W1)
    return out_p[:n]
